# baseline (speedup 1.0000x reference)
import math

import jax
import jax.numpy as jnp
from jax.experimental import pallas as pl
from jax.experimental.pallas import tpu as pltpu

_QLEN = 4
_L = 128
_B = 16
_NHID = 512
_DK = 256
_N = 10
_K = 4
_LB = 8
_ROWS = _QLEN * _B
_SCALE = 1.0 / math.sqrt(_DK)
_NEG = -3.0e38


def _cache_body(q_ref, w_ref, k_ref, b_ref, wout_ref, iout_ref, acc_ref):
    i = pl.program_id(0)

    @pl.when(i == 0)
    def _init():
        acc_ref[...] = jnp.zeros_like(acc_ref)

    part = jnp.zeros((_ROWS, _DK), jnp.float32)
    for j in range(_LB):
        qj = q_ref[:, j].reshape(_ROWS, _NHID)
        wj = w_ref[:, j]
        part = part + jax.lax.dot_general(
            qj, wj, (((1,), (1,)), ((), ())),
            preferred_element_type=jnp.float32)
    acc_ref[...] += part

    @pl.when(i == (_L // _LB) - 1)
    def _epilogue():
        qd = acc_ref[...] + b_ref[...]
        qd3 = qd.reshape(_QLEN, _B, _DK)
        cols = []
        for n in range(_N):
            kn = k_ref[n]
            cols.append(jnp.sum(qd3 * kn[None], axis=-1).reshape(_ROWS, 1))
        scores = jnp.concatenate(cols, axis=1) * _SCALE
        m = jnp.max(scores, axis=-1, keepdims=True)
        e = jnp.exp(scores - m)
        p = e / jnp.sum(e, axis=-1, keepdims=True)
        iota = jax.lax.broadcasted_iota(jnp.int32, (_ROWS, _N), 1)
        work = p
        vals = []
        for j in range(_K):
            mv = jnp.max(work, axis=-1, keepdims=True)
            sel = work == mv
            idx = jnp.min(jnp.where(sel, iota, _N), axis=-1)
            vals.append(mv)
            iout_ref[:, j:j + 1] = idx.astype(jnp.int32).reshape(_ROWS, 1)
            work = jnp.where(iota == idx[:, None], _NEG, work)
        w4 = jnp.concatenate(vals, axis=1)
        m2 = jnp.max(w4, axis=-1, keepdims=True)
        e2 = jnp.exp(w4 - m2)
        wout_ref[...] = e2 / jnp.sum(e2, axis=-1, keepdims=True)


def kernel(query, keys, values, W, b):
    del values
    w3 = W.reshape(_DK, _L, _NHID)
    b2 = b.reshape(1, _DK)
    wk, ik = pl.pallas_call(
        _cache_body,
        grid=(_L // _LB,),
        in_specs=[
            pl.BlockSpec((_QLEN, _LB, _B, _NHID), lambda i: (0, i, 0, 0)),
            pl.BlockSpec((_DK, _LB, _NHID), lambda i: (0, i, 0)),
            pl.BlockSpec((_N, _B, _DK), lambda i: (0, 0, 0)),
            pl.BlockSpec((1, _DK), lambda i: (0, 0)),
        ],
        out_specs=[
            pl.BlockSpec((_ROWS, _K), lambda i: (0, 0)),
            pl.BlockSpec((_ROWS, _K), lambda i: (0, 0)),
        ],
        out_shape=[
            jax.ShapeDtypeStruct((_ROWS, _K), jnp.float32),
            jax.ShapeDtypeStruct((_ROWS, _K), jnp.int32),
        ],
        scratch_shapes=[pltpu.VMEM((_ROWS, _DK), jnp.float32)],
        compiler_params=pltpu.CompilerParams(
            dimension_semantics=("arbitrary",),
        ),
    )(query, w3, keys, b2)
    return wk.reshape(_ROWS, 1, _K), ik.T
